# Initial kernel scaffold; baseline (speedup 1.0000x reference)
import jax
import jax.numpy as jnp
from jax.experimental import pallas as pl


def kernel(query_embedding, support_set_embeddings, padding_mask, Wq, Ws):
    raise NotImplementedError("write your pallas kernel here")


# speedup vs baseline: 1.3670x; 1.3670x over previous
import math

import jax
import jax.numpy as jnp
import numpy as np
from jax.experimental import pallas as pl
from jax.experimental.pallas import tpu as pltpu

_B, _N, _D, _H = 16, 4096, 1024, 16
_DH = _D // _H
_K = 128
_NB = 512
_NBLKS = _N // _NB
_MININT = np.int32(-(2 ** 31))
_LOW31 = np.int32(0x7FFFFFFF)


def _sortable(f32):
    i = jax.lax.bitcast_convert_type(f32, jnp.int32)
    return jnp.where(i >= 0, i, i ^ _LOW31)


def _unsortable(key):
    i = jnp.where(key >= 0, key, key ^ _LOW31)
    return jax.lax.bitcast_convert_type(i, jnp.float32)


def _sim_kernel(q_ref, maskT_ref, wq_ref, ws_ref, s_ref, out_ref,
                ut_ref, sim_ref):
    b = pl.program_id(0)
    nb = pl.program_id(1)

    @pl.when(nb == 0)
    def _fold_query():
        qv = q_ref[0]
        qn = qv / jnp.maximum(
            jnp.sqrt(jnp.sum(qv * qv, axis=1, keepdims=True)), 1e-8)
        qp = jax.lax.dot_general(
            qn, wq_ref[...], (((1,), (1,)), ((), ())),
            preferred_element_type=jnp.float32,
            precision=jax.lax.Precision.HIGHEST)
        d_iota = jax.lax.broadcasted_iota(jnp.int32, (_H, _D), 1)
        h_iota = jax.lax.broadcasted_iota(jnp.int32, (_H, _D), 0)
        bm = jnp.where((d_iota // _DH) == h_iota, qp, 0.0)
        ut_ref[...] = jax.lax.dot_general(
            bm, ws_ref[...], (((1,), (0,)), ((), ())),
            preferred_element_type=jnp.float32,
            precision=jax.lax.Precision.HIGHEST)

    s = s_ref[0]
    denom = jnp.maximum(
        jnp.sqrt(jnp.sum(s * s, axis=1, keepdims=True)), 1e-8)
    denom = denom * np.float32(math.sqrt(_DH))
    dots = jax.lax.dot_general(
        s, ut_ref[...], (((1,), (1,)), ((), ())),
        preferred_element_type=jnp.float32,
            precision=jax.lax.Precision.HIGHEST)
    simv = dots / denom
    lane = jax.lax.broadcasted_iota(jnp.int32, (_NB, _B), 1)
    mcol = jnp.max(jnp.where(lane == b, maskT_ref[...], 0.0),
                   axis=1, keepdims=True)
    simv = jnp.where(mcol > 0, simv, -jnp.inf)
    sim_ref[pl.ds(nb * _NB, _NB), :] = simv

    @pl.when(nb == _NBLKS - 1)
    def _select_and_reduce():
        sv = sim_ref[...]
        keys = _sortable(sv)
        prefix = jnp.zeros((1, _H), jnp.int32)
        for bit in range(31, -1, -1):
            bitc = _MININT if bit == 31 else np.int32(1 << bit)
            cand = prefix | bitc
            thresh = cand ^ _MININT
            cnt = jnp.sum((keys >= thresh).astype(jnp.int32),
                          axis=0, keepdims=True)
            prefix = jnp.where(cnt >= _K, cand, prefix)
        t_f = _unsortable(prefix ^ _MININT)
        mx = jnp.max(sv, axis=0, keepdims=True)
        gt = sv > t_f
        c_gt = jnp.sum(gt.astype(jnp.float32), axis=0, keepdims=True)
        sum_gt = jnp.sum(jnp.where(gt, jnp.exp(sv - mx), 0.0),
                         axis=0, keepdims=True)
        sum_exp = sum_gt + (_K - c_gt) * jnp.exp(t_f - mx)
        lse = mx + jnp.log(sum_exp)
        out_ref[...] = jnp.mean(lse, axis=1, keepdims=True).reshape(1, 1, 1)


def kernel(query_embedding, support_set_embeddings, padding_mask, Wq, Ws):
    maskT = padding_mask.T.astype(jnp.float32)
    out = pl.pallas_call(
        _sim_kernel,
        grid=(_B, _NBLKS),
        in_specs=[
            pl.BlockSpec((1, 1, _D), lambda b, nb: (b, 0, 0)),
            pl.BlockSpec((_NB, _B), lambda b, nb: (nb, 0)),
            pl.BlockSpec((_D, _D), lambda b, nb: (0, 0)),
            pl.BlockSpec((_D, _D), lambda b, nb: (0, 0)),
            pl.BlockSpec((1, _NB, _D), lambda b, nb: (b, nb, 0)),
        ],
        out_specs=pl.BlockSpec((1, 1, 1), lambda b, nb: (b, 0, 0)),
        out_shape=jax.ShapeDtypeStruct((_B, 1, 1), jnp.float32),
        scratch_shapes=[
            pltpu.VMEM((_H, _D), jnp.float32),
            pltpu.VMEM((_N, _H), jnp.float32),
        ],
        compiler_params=pltpu.CompilerParams(
            dimension_semantics=("arbitrary", "arbitrary"),
        ),
    )(query_embedding, maskT, Wq, Ws, support_set_embeddings)
    return out.reshape(_B, 1)
